# baseline (speedup 1.0000x reference)
import jax
import jax.numpy as jnp
from jax import lax
from jax.experimental import pallas as pl
from jax.experimental.pallas import tpu as pltpu
from jax.experimental.pallas import tpu_sc as plsc

N = 10000
E = 320000
D1 = 128
DC = 40
D2 = 128

NC = 2
NS = 16
NW = NC * NS
CHUNK = 128

NPAD = 10112
RPT = NPAD // NS

NCH = 80
EPW = NCH * CHUNK
EPAD = NW * EPW
SLOW_C = 1
QS = 62
QF = 2 * NCH - QS

_MESH = plsc.VectorSubcoreMesh(core_axis_name="c", subcore_axis_name="s")


def _deg_body(dst_hbm, out_hbm, didx_v, deg_v):
    c = lax.axis_index("c")
    s = lax.axis_index("s")
    wid = s * NC + c

    zero16 = jnp.zeros((16,), jnp.float32)
    ones = jnp.ones((16,), jnp.float32)

    def zb(i, carry):
        deg_v[pl.ds(i * 16, 16)] = zero16
        return carry

    lax.fori_loop(0, NPAD // 16, zb, 0)

    def chunk(j, carry):
        base = wid * EPW + j * CHUNK
        pltpu.sync_copy(dst_hbm.at[pl.ds(base, CHUNK)], didx_v)
        for k in range(CHUNK // 16):
            idx16 = didx_v[pl.ds(k * 16, 16)]
            plsc.addupdate_scatter(deg_v, [idx16], ones)
        return carry

    lax.fori_loop(0, NCH, chunk, 0)
    pltpu.sync_copy(deg_v, out_hbm.at[wid, 0])


_deg_kernel = pl.kernel(
    _deg_body,
    out_type=jax.ShapeDtypeStruct((NW, 8, NPAD), jnp.float32),
    mesh=_MESH,
    scratch_types=[
        pltpu.VMEM((CHUNK,), jnp.int32),
        pltpu.VMEM((NPAD,), jnp.float32),
    ],
    compiler_params=pltpu.CompilerParams(needs_layout_passes=False),
)


def _make_prop(d):
    def body(g_hbm, src_hbm, dst_hbm, z_hbm, out_hbm,
             sidx_a, didx_a, rows_a, acc, sem_ga):
        c = lax.axis_index("c")
        s = lax.axis_index("s")
        wid = s * NC + c
        r0 = s * RPT

        def row_chunks(fn):
            off = 0
            while off < RPT:
                cb = min(CHUNK, RPT - off)
                fn(off, cb)
                off += cb

        def init(off, cb):
            pltpu.sync_copy(z_hbm.at[pl.ds(r0 + off, cb)],
                            rows_a.at[pl.ds(0, cb)])
            pltpu.sync_copy(rows_a.at[pl.ds(0, cb)],
                            acc.at[pl.ds(r0 + off, cb)])

        row_chunks(init)
        plsc.subcore_barrier()

        nch_c = jnp.where(c == SLOW_C, QS, QF)
        start = jnp.where(c == SLOW_C, s * QS, NS * QS + s * QF)

        def chunk(j, carry):
            base = (start + j) * CHUNK
            pltpu.sync_copy(src_hbm.at[pl.ds(base, CHUNK)], sidx_a)
            pltpu.sync_copy(dst_hbm.at[pl.ds(base, CHUNK)], didx_a)
            pltpu.async_copy(g_hbm.at[sidx_a], rows_a, sem_ga).wait()
            pltpu.sync_copy(rows_a, acc.at[didx_a], add=True)
            return carry

        lax.fori_loop(0, nch_c, chunk, 0)
        plsc.subcore_barrier()

        def writeback(off, cb):
            pltpu.sync_copy(acc.at[pl.ds(r0 + off, cb)],
                            rows_a.at[pl.ds(0, cb)])
            pltpu.sync_copy(rows_a.at[pl.ds(0, cb)],
                            out_hbm.at[c, pl.ds(r0 + off, cb)])

        row_chunks(writeback)

    return pl.kernel(
        body,
        out_type=jax.ShapeDtypeStruct((NC, NPAD, d), jnp.float32),
        mesh=_MESH,
        scratch_types=[
            pltpu.VMEM((CHUNK,), jnp.int32),
            pltpu.VMEM((CHUNK,), jnp.int32),
            pltpu.VMEM((CHUNK, d), jnp.float32),
            pltpu.VMEM_SHARED((NPAD, d), jnp.float32),
            pltpu.SemaphoreType.DMA,
        ],
    )


_prop128 = _make_prop(D1)


def _stage_a_body(x_ref, w1_ref, degp_ref, g_ref, dinv_ref):
    deg0 = jnp.sum(degp_ref[...], axis=1, keepdims=True)
    dinv = lax.rsqrt(jnp.maximum(deg0, 1.0))
    dinv_ref[...] = dinv
    h = jnp.dot(x_ref[...], w1_ref[...], preferred_element_type=jnp.float32)
    g_ref[...] = h * dinv


_stage_a = pl.pallas_call(
    _stage_a_body,
    out_shape=[
        jax.ShapeDtypeStruct((NPAD, D1), jnp.float32),
        jax.ShapeDtypeStruct((NPAD, 1), jnp.float32),
    ],
)


def _stage_b_body(p_ref, dinv_ref, b1_ref, w2_ref, g2_ref):
    acc = p_ref[0] + p_ref[1]
    dinv = dinv_ref[...]
    h = jnp.maximum(acc * dinv + b1_ref[...], 0.0)
    g2_ref[...] = jnp.dot(h, w2_ref[...],
                          preferred_element_type=jnp.float32) * dinv


_stage_b = pl.pallas_call(
    _stage_b_body,
    out_shape=jax.ShapeDtypeStruct((NPAD, D2), jnp.float32),
)


def _stage_c_body(q_ref, dinv_ref, b2_ref, o_ref):
    acc = q_ref[0] + q_ref[1]
    logits = acc * dinv_ref[...] + b2_ref[...]
    col = lax.broadcasted_iota(jnp.int32, (NPAD, D2), 1)
    valid = col < DC
    logits = jnp.where(valid, logits, -jnp.inf)
    m = jnp.max(logits, axis=1, keepdims=True)
    ex = jnp.where(valid, jnp.exp(logits - m), 0.0)
    lse = jnp.log(jnp.sum(ex, axis=1, keepdims=True))
    out = logits - m - lse
    o_ref[...] = out[:N, :DC]


_stage_c = pl.pallas_call(
    _stage_c_body,
    out_shape=jax.ShapeDtypeStruct((N, DC), jnp.float32),
)


def kernel(inputs, edge_index, W1, b1, W2, b2, epoch):
    ei = edge_index.astype(jnp.int32)
    pad = jnp.full((EPAD - E,), N, dtype=jnp.int32)
    src = jnp.concatenate([ei[0], pad])
    dst = jnp.concatenate([ei[1], pad])

    x = jnp.concatenate(
        [inputs, jnp.zeros((NPAD - N, D1), jnp.float32)], axis=0)
    w2p = jnp.concatenate(
        [W2, jnp.zeros((D1, D2 - DC), jnp.float32)], axis=1)
    b1r = b1.reshape(1, D1)
    b2r = jnp.concatenate([b2, jnp.zeros((D2 - DC,), jnp.float32)]
                          ).reshape(1, D2)

    z128 = jnp.zeros((NPAD, D1), jnp.float32)

    degp = _deg_kernel(dst)
    degt = jnp.transpose(degp[:, 0, :])
    g1, dinv = _stage_a(x, W1, degt)
    p = _prop128(g1, src, dst, z128)
    g2 = _stage_b(p, dinv, b1r, w2p)
    q = _prop128(g2, src, dst, z128)
    return _stage_c(q, dinv, b2r)


# speedup vs baseline: 2.3877x; 2.3877x over previous
import jax
import jax.numpy as jnp
from jax import lax
from jax.experimental import pallas as pl
from jax.experimental.pallas import tpu as pltpu
from jax.experimental.pallas import tpu_sc as plsc

N = 10000
E = 320000
D1 = 128
DC = 40
D2 = 128

NC = 2
NS = 16
NW = NC * NS
CHUNK = 128

NPAD = 10112
RPT = NPAD // NS

NCH = 80
EPW = NCH * CHUNK
EPAD = NW * EPW

_MESH = plsc.VectorSubcoreMesh(core_axis_name="c", subcore_axis_name="s")


def _deg_body(dst_hbm, out_hbm, didx_v, deg_v):
    c = lax.axis_index("c")
    s = lax.axis_index("s")
    wid = s * NC + c

    zero16 = jnp.zeros((16,), jnp.float32)
    ones = jnp.ones((16,), jnp.float32)

    def zb(i, carry):
        deg_v[pl.ds(i * 16, 16)] = zero16
        return carry

    lax.fori_loop(0, NPAD // 16, zb, 0)

    def chunk(j, carry):
        base = wid * EPW + j * CHUNK
        pltpu.sync_copy(dst_hbm.at[pl.ds(base, CHUNK)], didx_v)
        for k in range(CHUNK // 16):
            idx16 = didx_v[pl.ds(k * 16, 16)]
            plsc.addupdate_scatter(deg_v, [idx16], ones)
        return carry

    lax.fori_loop(0, NCH, chunk, 0)
    pltpu.sync_copy(deg_v, out_hbm.at[wid, 0])


_deg_kernel = pl.kernel(
    _deg_body,
    out_type=jax.ShapeDtypeStruct((NW, 8, NPAD), jnp.float32),
    mesh=_MESH,
    scratch_types=[
        pltpu.VMEM((CHUNK,), jnp.int32),
        pltpu.VMEM((NPAD,), jnp.float32),
    ],
    compiler_params=pltpu.CompilerParams(needs_layout_passes=False),
)


def _make_prop(d):
    def body(g_hbm, src_hbm, dst_hbm, z_hbm, out_hbm,
             sidx_a, didx_a, rows_a, acc, sem_ga):
        c = lax.axis_index("c")
        s = lax.axis_index("s")
        wid = s * NC + c
        r0 = s * RPT

        def row_chunks(fn):
            off = 0
            while off < RPT:
                cb = min(CHUNK, RPT - off)
                fn(off, cb)
                off += cb

        def init(off, cb):
            pltpu.sync_copy(z_hbm.at[pl.ds(r0 + off, cb)],
                            rows_a.at[pl.ds(0, cb)])
            pltpu.sync_copy(rows_a.at[pl.ds(0, cb)],
                            acc.at[pl.ds(r0 + off, cb)])

        row_chunks(init)
        plsc.subcore_barrier()

        def chunk(j, carry):
            base = wid * EPW + j * CHUNK
            pltpu.sync_copy(src_hbm.at[pl.ds(base, CHUNK)], sidx_a)
            pltpu.sync_copy(dst_hbm.at[pl.ds(base, CHUNK)], didx_a)
            pltpu.async_copy(g_hbm.at[sidx_a], rows_a, sem_ga).wait()
            pltpu.sync_copy(rows_a, acc.at[didx_a], add=True)
            return carry

        lax.fori_loop(0, NCH, chunk, 0)
        plsc.subcore_barrier()

        def writeback(off, cb):
            pltpu.sync_copy(acc.at[pl.ds(r0 + off, cb)],
                            rows_a.at[pl.ds(0, cb)])
            pltpu.sync_copy(rows_a.at[pl.ds(0, cb)],
                            out_hbm.at[c, pl.ds(r0 + off, cb)])

        row_chunks(writeback)

    return pl.kernel(
        body,
        out_type=jax.ShapeDtypeStruct((NC, NPAD, d), jnp.float32),
        mesh=_MESH,
        scratch_types=[
            pltpu.VMEM((CHUNK,), jnp.int32),
            pltpu.VMEM((CHUNK,), jnp.int32),
            pltpu.VMEM((CHUNK, d), jnp.float32),
            pltpu.VMEM_SHARED((NPAD, d), jnp.float32),
            pltpu.SemaphoreType.DMA,
        ],
    )


_prop128 = _make_prop(D1)


def _stage_a_body(x_ref, w1_ref, degp_ref, g_ref, dinv_ref):
    deg0 = jnp.sum(degp_ref[...], axis=1, keepdims=True)
    dinv = lax.rsqrt(jnp.maximum(deg0, 1.0))
    dinv_ref[...] = dinv
    h = jnp.dot(x_ref[...], w1_ref[...], preferred_element_type=jnp.float32)
    g_ref[...] = h * dinv


_stage_a = pl.pallas_call(
    _stage_a_body,
    out_shape=[
        jax.ShapeDtypeStruct((NPAD, D1), jnp.float32),
        jax.ShapeDtypeStruct((NPAD, 1), jnp.float32),
    ],
)


def _stage_b_body(p_ref, dinv_ref, b1_ref, w2_ref, g2_ref):
    acc = p_ref[0] + p_ref[1]
    dinv = dinv_ref[...]
    h = jnp.maximum(acc * dinv + b1_ref[...], 0.0)
    g2_ref[...] = jnp.dot(h, w2_ref[...],
                          preferred_element_type=jnp.float32) * dinv


_stage_b = pl.pallas_call(
    _stage_b_body,
    out_shape=jax.ShapeDtypeStruct((NPAD, D2), jnp.float32),
)


def _stage_c_body(q_ref, dinv_ref, b2_ref, o_ref):
    acc = q_ref[0] + q_ref[1]
    logits = acc * dinv_ref[...] + b2_ref[...]
    col = lax.broadcasted_iota(jnp.int32, (NPAD, D2), 1)
    valid = col < DC
    logits = jnp.where(valid, logits, -jnp.inf)
    m = jnp.max(logits, axis=1, keepdims=True)
    ex = jnp.where(valid, jnp.exp(logits - m), 0.0)
    lse = jnp.log(jnp.sum(ex, axis=1, keepdims=True))
    out = logits - m - lse
    o_ref[...] = out[:N, :DC]


_stage_c = pl.pallas_call(
    _stage_c_body,
    out_shape=jax.ShapeDtypeStruct((N, DC), jnp.float32),
)


def kernel(inputs, edge_index, W1, b1, W2, b2, epoch):
    ei = edge_index.astype(jnp.int32)
    pad = N + jnp.arange(EPAD - E, dtype=jnp.int32) % (NPAD - N)
    src = jnp.concatenate([ei[0], pad])
    dst = jnp.concatenate([ei[1], pad])

    x = jnp.concatenate(
        [inputs, jnp.zeros((NPAD - N, D1), jnp.float32)], axis=0)
    w2p = jnp.concatenate(
        [W2, jnp.zeros((D1, D2 - DC), jnp.float32)], axis=1)
    b1r = b1.reshape(1, D1)
    b2r = jnp.concatenate([b2, jnp.zeros((D2 - DC,), jnp.float32)]
                          ).reshape(1, D2)

    z128 = jnp.zeros((NPAD, D1), jnp.float32)

    degp = _deg_kernel(dst)
    degt = jnp.transpose(degp[:, 0, :])
    g1, dinv = _stage_a(x, W1, degt)
    p = _prop128(g1, src, dst, z128)
    g2 = _stage_b(p, dinv, b1r, w2p)
    q = _prop128(g2, src, dst, z128)
    return _stage_c(q, dinv, b2r)


# speedup vs baseline: 3.1221x; 1.3076x over previous
import jax
import jax.numpy as jnp
from jax import lax
from jax.experimental import pallas as pl
from jax.experimental.pallas import tpu as pltpu
from jax.experimental.pallas import tpu_sc as plsc

N = 10000
E = 320000
D1 = 128
DC = 40
D2 = 128

NC = 2
NS = 16
NW = NC * NS
CHUNK = 128

NPAD = 10112
RPT = NPAD // NS

NCH = 80
EPW = NCH * CHUNK
EPAD = NW * EPW

_MESH = plsc.VectorSubcoreMesh(core_axis_name="c", subcore_axis_name="s")


def _deg_body(dst_hbm, out_hbm, didx_v, deg_v):
    c = lax.axis_index("c")
    s = lax.axis_index("s")
    wid = s * NC + c

    zero16 = jnp.zeros((16,), jnp.float32)
    ones = jnp.ones((16,), jnp.float32)

    def zb(i, carry):
        deg_v[pl.ds(i * 16, 16)] = zero16
        return carry

    lax.fori_loop(0, NPAD // 16, zb, 0)

    def chunk(j, carry):
        base = wid * EPW + j * CHUNK
        pltpu.sync_copy(dst_hbm.at[pl.ds(base, CHUNK)], didx_v)
        for k in range(CHUNK // 16):
            idx16 = didx_v[pl.ds(k * 16, 16)]
            plsc.addupdate_scatter(deg_v, [idx16], ones)
        return carry

    lax.fori_loop(0, NCH, chunk, 0)
    pltpu.sync_copy(deg_v, out_hbm.at[wid, 0])


_deg_kernel = pl.kernel(
    _deg_body,
    out_type=jax.ShapeDtypeStruct((NW, 8, NPAD), jnp.float32),
    mesh=_MESH,
    scratch_types=[
        pltpu.VMEM((CHUNK,), jnp.int32),
        pltpu.VMEM((NPAD,), jnp.float32),
    ],
    compiler_params=pltpu.CompilerParams(needs_layout_passes=False),
)


def _make_prop(d):
    def body(g_hbm, src_hbm, dst_hbm, z_hbm, out_hbm,
             sidx_a, didx_a, sidx_b, didx_b, rows_a, rows_b, acc,
             sem_ga, sem_gb, sem_sa, sem_sb):
        c = lax.axis_index("c")
        s = lax.axis_index("s")
        wid = s * NC + c
        r0 = s * RPT

        def row_chunks(fn):
            off = 0
            while off < RPT:
                cb = min(CHUNK, RPT - off)
                fn(off, cb)
                off += cb

        def init(off, cb):
            pltpu.sync_copy(z_hbm.at[pl.ds(r0 + off, cb)],
                            rows_a.at[pl.ds(0, cb)])
            pltpu.sync_copy(rows_a.at[pl.ds(0, cb)],
                            acc.at[pl.ds(r0 + off, cb)])

        row_chunks(init)
        plsc.subcore_barrier()

        def pair(g, carry):
            base_a = wid * EPW + 2 * g * CHUNK
            base_b = base_a + CHUNK
            pltpu.sync_copy(src_hbm.at[pl.ds(base_a, CHUNK)], sidx_a)
            pltpu.sync_copy(dst_hbm.at[pl.ds(base_a, CHUNK)], didx_a)
            ga = pltpu.async_copy(g_hbm.at[sidx_a], rows_a, sem_ga)
            pltpu.sync_copy(src_hbm.at[pl.ds(base_b, CHUNK)], sidx_b)
            pltpu.sync_copy(dst_hbm.at[pl.ds(base_b, CHUNK)], didx_b)
            gb = pltpu.async_copy(g_hbm.at[sidx_b], rows_b, sem_gb)
            ga.wait()
            sa = pltpu.async_copy(rows_a, acc.at[didx_a], sem_sa, add=True)
            gb.wait()
            sb = pltpu.async_copy(rows_b, acc.at[didx_b], sem_sb, add=True)
            sa.wait()
            sb.wait()
            return carry

        lax.fori_loop(0, NCH // 2, pair, 0)
        plsc.subcore_barrier()

        def writeback(off, cb):
            pltpu.sync_copy(acc.at[pl.ds(r0 + off, cb)],
                            rows_a.at[pl.ds(0, cb)])
            pltpu.sync_copy(rows_a.at[pl.ds(0, cb)],
                            out_hbm.at[c, pl.ds(r0 + off, cb)])

        row_chunks(writeback)

    return pl.kernel(
        body,
        out_type=jax.ShapeDtypeStruct((NC, NPAD, d), jnp.float32),
        mesh=_MESH,
        scratch_types=[
            pltpu.VMEM((CHUNK,), jnp.int32),
            pltpu.VMEM((CHUNK,), jnp.int32),
            pltpu.VMEM((CHUNK,), jnp.int32),
            pltpu.VMEM((CHUNK,), jnp.int32),
            pltpu.VMEM((CHUNK, d), jnp.float32),
            pltpu.VMEM((CHUNK, d), jnp.float32),
            pltpu.VMEM_SHARED((NPAD, d), jnp.float32),
            pltpu.SemaphoreType.DMA,
            pltpu.SemaphoreType.DMA,
            pltpu.SemaphoreType.DMA,
            pltpu.SemaphoreType.DMA,
        ],
    )


_prop128 = _make_prop(D1)


def _stage_a_body(x_ref, w1_ref, degp_ref, g_ref, dinv_ref):
    deg0 = jnp.sum(degp_ref[...], axis=1, keepdims=True)
    dinv = lax.rsqrt(jnp.maximum(deg0, 1.0))
    dinv_ref[...] = dinv
    h = jnp.dot(x_ref[...], w1_ref[...], preferred_element_type=jnp.float32)
    g_ref[...] = h * dinv


_stage_a = pl.pallas_call(
    _stage_a_body,
    out_shape=[
        jax.ShapeDtypeStruct((NPAD, D1), jnp.float32),
        jax.ShapeDtypeStruct((NPAD, 1), jnp.float32),
    ],
)


def _stage_b_body(p_ref, dinv_ref, b1_ref, w2_ref, g2_ref):
    acc = p_ref[0] + p_ref[1]
    dinv = dinv_ref[...]
    h = jnp.maximum(acc * dinv + b1_ref[...], 0.0)
    g2_ref[...] = jnp.dot(h, w2_ref[...],
                          preferred_element_type=jnp.float32) * dinv


_stage_b = pl.pallas_call(
    _stage_b_body,
    out_shape=jax.ShapeDtypeStruct((NPAD, D2), jnp.float32),
)


def _stage_c_body(q_ref, dinv_ref, b2_ref, o_ref):
    acc = q_ref[0] + q_ref[1]
    logits = acc * dinv_ref[...] + b2_ref[...]
    col = lax.broadcasted_iota(jnp.int32, (NPAD, D2), 1)
    valid = col < DC
    logits = jnp.where(valid, logits, -jnp.inf)
    m = jnp.max(logits, axis=1, keepdims=True)
    ex = jnp.where(valid, jnp.exp(logits - m), 0.0)
    lse = jnp.log(jnp.sum(ex, axis=1, keepdims=True))
    out = logits - m - lse
    o_ref[...] = out[:N, :DC]


_stage_c = pl.pallas_call(
    _stage_c_body,
    out_shape=jax.ShapeDtypeStruct((N, DC), jnp.float32),
)


def kernel(inputs, edge_index, W1, b1, W2, b2, epoch):
    ei = edge_index.astype(jnp.int32)
    pad = N + jnp.arange(EPAD - E, dtype=jnp.int32) % (NPAD - N)
    src = jnp.concatenate([ei[0], pad])
    dst = jnp.concatenate([ei[1], pad])

    x = jnp.concatenate(
        [inputs, jnp.zeros((NPAD - N, D1), jnp.float32)], axis=0)
    w2p = jnp.concatenate(
        [W2, jnp.zeros((D1, D2 - DC), jnp.float32)], axis=1)
    b1r = b1.reshape(1, D1)
    b2r = jnp.concatenate([b2, jnp.zeros((D2 - DC,), jnp.float32)]
                          ).reshape(1, D2)

    z128 = jnp.zeros((NPAD, D1), jnp.float32)

    degp = _deg_kernel(dst)
    degt = jnp.transpose(degp[:, 0, :])
    g1, dinv = _stage_a(x, W1, degt)
    p = _prop128(g1, src, dst, z128)
    g2 = _stage_b(p, dinv, b1r, w2p)
    q = _prop128(g2, src, dst, z128)
    return _stage_c(q, dinv, b2r)


# speedup vs baseline: 3.4112x; 1.0926x over previous
import jax
import jax.numpy as jnp
from jax import lax
from jax.experimental import pallas as pl
from jax.experimental.pallas import tpu as pltpu
from jax.experimental.pallas import tpu_sc as plsc

N = 10000
E = 320000
D1 = 128
DC = 40
D2 = 128

NC = 2
NS = 16
NW = NC * NS
CHUNK = 128

NPAD = 10112
RPT = NPAD // NS

NCH = 81
EPW = NCH * CHUNK
EPAD = NW * EPW

_MESH = plsc.VectorSubcoreMesh(core_axis_name="c", subcore_axis_name="s")


def _deg_body(dst_hbm, out_hbm, didx_v, deg_v):
    c = lax.axis_index("c")
    s = lax.axis_index("s")
    wid = s * NC + c

    zero16 = jnp.zeros((16,), jnp.float32)
    ones = jnp.ones((16,), jnp.float32)

    def zb(i, carry):
        deg_v[pl.ds(i * 16, 16)] = zero16
        return carry

    lax.fori_loop(0, NPAD // 16, zb, 0)

    def chunk(j, carry):
        base = wid * EPW + j * CHUNK
        pltpu.sync_copy(dst_hbm.at[pl.ds(base, CHUNK)], didx_v)
        for k in range(CHUNK // 16):
            idx16 = didx_v[pl.ds(k * 16, 16)]
            plsc.addupdate_scatter(deg_v, [idx16], ones)
        return carry

    lax.fori_loop(0, NCH, chunk, 0)
    pltpu.sync_copy(deg_v, out_hbm.at[wid, 0])


_deg_kernel = pl.kernel(
    _deg_body,
    out_type=jax.ShapeDtypeStruct((NW, 8, NPAD), jnp.float32),
    mesh=_MESH,
    scratch_types=[
        pltpu.VMEM((CHUNK,), jnp.int32),
        pltpu.VMEM((NPAD,), jnp.float32),
    ],
    compiler_params=pltpu.CompilerParams(needs_layout_passes=False),
)


def _make_prop(d):
    def body(g_hbm, src_hbm, dst_hbm, z_hbm, out_hbm,
             sidx_a, didx_a, sidx_b, didx_b, sidx_c, didx_c,
             rows_a, rows_b, rows_c, acc,
             sem_ga, sem_gb, sem_gc, sem_sa, sem_sb, sem_sc):
        c = lax.axis_index("c")
        s = lax.axis_index("s")
        wid = s * NC + c
        r0 = s * RPT

        def row_chunks(fn):
            off = 0
            while off < RPT:
                cb = min(CHUNK, RPT - off)
                fn(off, cb)
                off += cb

        def init(off, cb):
            pltpu.sync_copy(z_hbm.at[pl.ds(r0 + off, cb)],
                            rows_a.at[pl.ds(0, cb)])
            pltpu.sync_copy(rows_a.at[pl.ds(0, cb)],
                            acc.at[pl.ds(r0 + off, cb)])

        row_chunks(init)
        plsc.subcore_barrier()

        def triple(g, carry):
            base_a = wid * EPW + 3 * g * CHUNK
            base_b = base_a + CHUNK
            base_c = base_b + CHUNK
            pltpu.sync_copy(src_hbm.at[pl.ds(base_a, CHUNK)], sidx_a)
            pltpu.sync_copy(dst_hbm.at[pl.ds(base_a, CHUNK)], didx_a)
            ga = pltpu.async_copy(g_hbm.at[sidx_a], rows_a, sem_ga)
            pltpu.sync_copy(src_hbm.at[pl.ds(base_b, CHUNK)], sidx_b)
            pltpu.sync_copy(dst_hbm.at[pl.ds(base_b, CHUNK)], didx_b)
            gb = pltpu.async_copy(g_hbm.at[sidx_b], rows_b, sem_gb)
            pltpu.sync_copy(src_hbm.at[pl.ds(base_c, CHUNK)], sidx_c)
            pltpu.sync_copy(dst_hbm.at[pl.ds(base_c, CHUNK)], didx_c)
            gc = pltpu.async_copy(g_hbm.at[sidx_c], rows_c, sem_gc)
            ga.wait()
            sa = pltpu.async_copy(rows_a, acc.at[didx_a], sem_sa, add=True)
            gb.wait()
            sb = pltpu.async_copy(rows_b, acc.at[didx_b], sem_sb, add=True)
            gc.wait()
            sc = pltpu.async_copy(rows_c, acc.at[didx_c], sem_sc, add=True)
            sa.wait()
            sb.wait()
            sc.wait()
            return carry

        lax.fori_loop(0, NCH // 3, triple, 0)
        plsc.subcore_barrier()

        def writeback(off, cb):
            pltpu.sync_copy(acc.at[pl.ds(r0 + off, cb)],
                            rows_a.at[pl.ds(0, cb)])
            pltpu.sync_copy(rows_a.at[pl.ds(0, cb)],
                            out_hbm.at[c, pl.ds(r0 + off, cb)])

        row_chunks(writeback)

    return pl.kernel(
        body,
        out_type=jax.ShapeDtypeStruct((NC, NPAD, d), jnp.float32),
        mesh=_MESH,
        scratch_types=(
            [pltpu.VMEM((CHUNK,), jnp.int32)] * 6
            + [pltpu.VMEM((CHUNK, d), jnp.float32)] * 3
            + [pltpu.VMEM_SHARED((NPAD, d), jnp.float32)]
            + [pltpu.SemaphoreType.DMA] * 6
        ),
    )


_prop128 = _make_prop(D1)


def _stage_a_body(x_ref, w1_ref, degp_ref, g_ref, dinv_ref):
    deg0 = jnp.sum(degp_ref[...], axis=1, keepdims=True)
    dinv = lax.rsqrt(jnp.maximum(deg0, 1.0))
    dinv_ref[...] = dinv
    h = jnp.dot(x_ref[...], w1_ref[...], preferred_element_type=jnp.float32)
    g_ref[...] = h * dinv


_stage_a = pl.pallas_call(
    _stage_a_body,
    out_shape=[
        jax.ShapeDtypeStruct((NPAD, D1), jnp.float32),
        jax.ShapeDtypeStruct((NPAD, 1), jnp.float32),
    ],
)


def _stage_b_body(p_ref, dinv_ref, b1_ref, w2_ref, g2_ref):
    acc = p_ref[0] + p_ref[1]
    dinv = dinv_ref[...]
    h = jnp.maximum(acc * dinv + b1_ref[...], 0.0)
    g2_ref[...] = jnp.dot(h, w2_ref[...],
                          preferred_element_type=jnp.float32) * dinv


_stage_b = pl.pallas_call(
    _stage_b_body,
    out_shape=jax.ShapeDtypeStruct((NPAD, D2), jnp.float32),
)


def _stage_c_body(q_ref, dinv_ref, b2_ref, o_ref):
    acc = q_ref[0] + q_ref[1]
    logits = acc * dinv_ref[...] + b2_ref[...]
    col = lax.broadcasted_iota(jnp.int32, (NPAD, D2), 1)
    valid = col < DC
    logits = jnp.where(valid, logits, -jnp.inf)
    m = jnp.max(logits, axis=1, keepdims=True)
    ex = jnp.where(valid, jnp.exp(logits - m), 0.0)
    lse = jnp.log(jnp.sum(ex, axis=1, keepdims=True))
    out = logits - m - lse
    o_ref[...] = out[:N, :DC]


_stage_c = pl.pallas_call(
    _stage_c_body,
    out_shape=jax.ShapeDtypeStruct((N, DC), jnp.float32),
)


def kernel(inputs, edge_index, W1, b1, W2, b2, epoch):
    ei = edge_index.astype(jnp.int32)
    pad = N + jnp.arange(EPAD - E, dtype=jnp.int32) % (NPAD - N)
    src = jnp.concatenate([ei[0], pad])
    dst = jnp.concatenate([ei[1], pad])

    x = jnp.concatenate(
        [inputs, jnp.zeros((NPAD - N, D1), jnp.float32)], axis=0)
    w2p = jnp.concatenate(
        [W2, jnp.zeros((D1, D2 - DC), jnp.float32)], axis=1)
    b1r = b1.reshape(1, D1)
    b2r = jnp.concatenate([b2, jnp.zeros((D2 - DC,), jnp.float32)]
                          ).reshape(1, D2)

    z128 = jnp.zeros((NPAD, D1), jnp.float32)

    degp = _deg_kernel(dst)
    degt = jnp.transpose(degp[:, 0, :])
    g1, dinv = _stage_a(x, W1, degt)
    p = _prop128(g1, src, dst, z128)
    g2 = _stage_b(p, dinv, b1r, w2p)
    q = _prop128(g2, src, dst, z128)
    return _stage_c(q, dinv, b2r)


# speedup vs baseline: 3.7579x; 1.1016x over previous
import jax
import jax.numpy as jnp
from jax import lax
from jax.experimental import pallas as pl
from jax.experimental.pallas import tpu as pltpu
from jax.experimental.pallas import tpu_sc as plsc

N = 10000
E = 320000
D1 = 128
DC = 40
D2 = 128

NC = 2
NS = 16
NW = NC * NS
CHUNK = 128

NPAD = 10112
RPT = NPAD // NS

NCH = 81
EPW = NCH * CHUNK
EPAD = NW * EPW

_MESH = plsc.VectorSubcoreMesh(core_axis_name="c", subcore_axis_name="s")


def _deg_body(eidx_hbm, out_hbm, didx_v, deg_v):
    c = lax.axis_index("c")
    s = lax.axis_index("s")
    wid = s * NC + c

    zero16 = jnp.zeros((16,), jnp.float32)
    ones = jnp.ones((16,), jnp.float32)

    def zb(i, carry):
        deg_v[pl.ds(i * 16, 16)] = zero16
        return carry

    lax.fori_loop(0, NPAD // 16, zb, 0)

    def chunk(j, carry):
        pltpu.sync_copy(eidx_hbm.at[wid * NCH + j, 1], didx_v)
        for k in range(CHUNK // 16):
            idx16 = didx_v[pl.ds(k * 16, 16)]
            plsc.addupdate_scatter(deg_v, [idx16], ones)
        return carry

    lax.fori_loop(0, NCH, chunk, 0)
    pltpu.sync_copy(deg_v, out_hbm.at[wid, 0])


_deg_kernel = pl.kernel(
    _deg_body,
    out_type=jax.ShapeDtypeStruct((NW, 8, NPAD), jnp.float32),
    mesh=_MESH,
    scratch_types=[
        pltpu.VMEM((CHUNK,), jnp.int32),
        pltpu.VMEM((NPAD,), jnp.float32),
    ],
    compiler_params=pltpu.CompilerParams(needs_layout_passes=False),
)


def _make_prop(d):
    def body(g_hbm, eidx_hbm, z_hbm, out_hbm,
             idx_a, idx_b, idx_c,
             rows_a, rows_b, rows_c, acc,
             sem_ga, sem_gb, sem_gc, sem_sa, sem_sb, sem_sc):
        c = lax.axis_index("c")
        s = lax.axis_index("s")
        wid = s * NC + c
        r0 = s * RPT

        def row_chunks(fn):
            off = 0
            while off < RPT:
                cb = min(CHUNK, RPT - off)
                fn(off, cb)
                off += cb

        def init(off, cb):
            pltpu.sync_copy(z_hbm.at[pl.ds(r0 + off, cb)],
                            rows_a.at[pl.ds(0, cb)])
            pltpu.sync_copy(rows_a.at[pl.ds(0, cb)],
                            acc.at[pl.ds(r0 + off, cb)])

        row_chunks(init)
        plsc.subcore_barrier()

        def triple(g, carry):
            row_a = wid * NCH + 3 * g
            pltpu.sync_copy(eidx_hbm.at[row_a], idx_a)
            ga = pltpu.async_copy(g_hbm.at[idx_a.at[0]], rows_a, sem_ga)
            pltpu.sync_copy(eidx_hbm.at[row_a + 1], idx_b)
            gb = pltpu.async_copy(g_hbm.at[idx_b.at[0]], rows_b, sem_gb)
            pltpu.sync_copy(eidx_hbm.at[row_a + 2], idx_c)
            gc = pltpu.async_copy(g_hbm.at[idx_c.at[0]], rows_c, sem_gc)
            ga.wait()
            sa = pltpu.async_copy(rows_a, acc.at[idx_a.at[1]], sem_sa, add=True)
            gb.wait()
            sb = pltpu.async_copy(rows_b, acc.at[idx_b.at[1]], sem_sb, add=True)
            gc.wait()
            sc = pltpu.async_copy(rows_c, acc.at[idx_c.at[1]], sem_sc, add=True)
            sa.wait()
            sb.wait()
            sc.wait()
            return carry

        lax.fori_loop(0, NCH // 3, triple, 0)
        plsc.subcore_barrier()

        def writeback(off, cb):
            pltpu.sync_copy(acc.at[pl.ds(r0 + off, cb)],
                            rows_a.at[pl.ds(0, cb)])
            pltpu.sync_copy(rows_a.at[pl.ds(0, cb)],
                            out_hbm.at[c, pl.ds(r0 + off, cb)])

        row_chunks(writeback)

    return pl.kernel(
        body,
        out_type=jax.ShapeDtypeStruct((NC, NPAD, d), jnp.float32),
        mesh=_MESH,
        scratch_types=(
            [pltpu.VMEM((2, CHUNK), jnp.int32)] * 3
            + [pltpu.VMEM((CHUNK, d), jnp.float32)] * 3
            + [pltpu.VMEM_SHARED((NPAD, d), jnp.float32)]
            + [pltpu.SemaphoreType.DMA] * 6
        ),
    )


_prop128 = _make_prop(D1)


def _stage_a_body(x_ref, w1_ref, degp_ref, g_ref, dinv_ref):
    deg0 = jnp.sum(degp_ref[...], axis=1, keepdims=True)
    dinv = lax.rsqrt(jnp.maximum(deg0, 1.0))
    dinv_ref[...] = dinv
    h = jnp.dot(x_ref[...], w1_ref[...], preferred_element_type=jnp.float32)
    g_ref[...] = h * dinv


_stage_a = pl.pallas_call(
    _stage_a_body,
    out_shape=[
        jax.ShapeDtypeStruct((NPAD, D1), jnp.float32),
        jax.ShapeDtypeStruct((NPAD, 1), jnp.float32),
    ],
)


def _stage_b_body(p_ref, dinv_ref, b1_ref, w2_ref, g2_ref):
    acc = p_ref[0] + p_ref[1]
    dinv = dinv_ref[...]
    h = jnp.maximum(acc * dinv + b1_ref[...], 0.0)
    g2_ref[...] = jnp.dot(h, w2_ref[...],
                          preferred_element_type=jnp.float32) * dinv


_stage_b = pl.pallas_call(
    _stage_b_body,
    out_shape=jax.ShapeDtypeStruct((NPAD, D2), jnp.float32),
)


def _stage_c_body(q_ref, dinv_ref, b2_ref, o_ref):
    acc = q_ref[0] + q_ref[1]
    logits = acc * dinv_ref[...] + b2_ref[...]
    col = lax.broadcasted_iota(jnp.int32, (NPAD, D2), 1)
    valid = col < DC
    logits = jnp.where(valid, logits, -jnp.inf)
    m = jnp.max(logits, axis=1, keepdims=True)
    ex = jnp.where(valid, jnp.exp(logits - m), 0.0)
    lse = jnp.log(jnp.sum(ex, axis=1, keepdims=True))
    out = logits - m - lse
    o_ref[...] = out[:N, :DC]


_stage_c = pl.pallas_call(
    _stage_c_body,
    out_shape=jax.ShapeDtypeStruct((N, DC), jnp.float32),
)


def kernel(inputs, edge_index, W1, b1, W2, b2, epoch):
    ei = edge_index.astype(jnp.int32)
    pad = N + jnp.arange(EPAD - E, dtype=jnp.int32) % (NPAD - N)
    src = jnp.concatenate([ei[0], pad])
    dst = jnp.concatenate([ei[1], pad])
    eidx = jnp.stack([src.reshape(NW * NCH, CHUNK),
                      dst.reshape(NW * NCH, CHUNK)], axis=1)

    x = jnp.concatenate(
        [inputs, jnp.zeros((NPAD - N, D1), jnp.float32)], axis=0)
    w2p = jnp.concatenate(
        [W2, jnp.zeros((D1, D2 - DC), jnp.float32)], axis=1)
    b1r = b1.reshape(1, D1)
    b2r = jnp.concatenate([b2, jnp.zeros((D2 - DC,), jnp.float32)]
                          ).reshape(1, D2)

    z128 = jnp.zeros((NPAD, D1), jnp.float32)

    degp = _deg_kernel(eidx)
    degt = jnp.transpose(degp[:, 0, :])
    g1, dinv = _stage_a(x, W1, degt)
    p = _prop128(g1, eidx, z128)
    g2 = _stage_b(p, dinv, b1r, w2p)
    q = _prop128(g2, eidx, z128)
    return _stage_c(q, dinv, b2r)


# speedup vs baseline: 3.9578x; 1.0532x over previous
import jax
import jax.numpy as jnp
from jax import lax
from jax.experimental import pallas as pl
from jax.experimental.pallas import tpu as pltpu
from jax.experimental.pallas import tpu_sc as plsc

N = 10000
E = 320000
D1 = 128
DC = 40
D2 = 128

NC = 2
NS = 16
NW = NC * NS
CHUNK = 128

NPAD = 10112
RPT = NPAD // NS

NCH = 81
EPW = NCH * CHUNK
EPAD = NW * EPW

_MESH = plsc.VectorSubcoreMesh(core_axis_name="c", subcore_axis_name="s")


def _deg_body(eidx_hbm, out_hbm, didx_a, didx_b, deg_v, sem_a, sem_b):
    c = lax.axis_index("c")
    s = lax.axis_index("s")
    wid = s * NC + c
    row0 = wid * NCH

    zero16 = jnp.zeros((16,), jnp.float32)
    ones = jnp.ones((16,), jnp.float32)

    def zb(i, carry):
        deg_v[pl.ds(i * 16, 16)] = zero16
        return carry

    lax.fori_loop(0, NPAD // 16, zb, 0)

    def scat(didx):
        for k in range(CHUNK // 16):
            idx16 = didx[pl.ds(k * 16, 16)]
            plsc.addupdate_scatter(deg_v, [idx16], ones)

    def pair(g, carry):
        j = row0 + 2 * g
        la = pltpu.async_copy(eidx_hbm.at[j, 1], didx_a, sem_a)
        lb = pltpu.async_copy(eidx_hbm.at[j + 1, 1], didx_b, sem_b)
        la.wait()
        scat(didx_a)
        lb.wait()
        scat(didx_b)
        return carry

    lax.fori_loop(0, NCH // 2, pair, 0)
    pltpu.sync_copy(eidx_hbm.at[row0 + NCH - 1, 1], didx_a)
    scat(didx_a)
    pltpu.sync_copy(deg_v, out_hbm.at[wid, 0])


_deg_kernel = pl.kernel(
    _deg_body,
    out_type=jax.ShapeDtypeStruct((NW, 8, NPAD), jnp.float32),
    mesh=_MESH,
    scratch_types=[
        pltpu.VMEM((CHUNK,), jnp.int32),
        pltpu.VMEM((CHUNK,), jnp.int32),
        pltpu.VMEM((NPAD,), jnp.float32),
        pltpu.SemaphoreType.DMA,
        pltpu.SemaphoreType.DMA,
    ],
    compiler_params=pltpu.CompilerParams(needs_layout_passes=False),
)


def _make_prop(d):
    def body(g_hbm, eidx_hbm, z_hbm, out_hbm,
             idx_a, idx_b, idx_c,
             rows_a, rows_b, rows_c, acc,
             sem_ga, sem_gb, sem_gc, sem_sa, sem_sb, sem_sc):
        c = lax.axis_index("c")
        s = lax.axis_index("s")
        wid = s * NC + c
        r0 = s * RPT

        def row_chunks(fn):
            off = 0
            while off < RPT:
                cb = min(CHUNK, RPT - off)
                fn(off, cb)
                off += cb

        def init(off, cb):
            pltpu.sync_copy(z_hbm.at[pl.ds(r0 + off, cb)],
                            rows_a.at[pl.ds(0, cb)])
            pltpu.sync_copy(rows_a.at[pl.ds(0, cb)],
                            acc.at[pl.ds(r0 + off, cb)])

        row_chunks(init)
        plsc.subcore_barrier()

        def triple(g, carry):
            row_a = wid * NCH + 3 * g
            pltpu.sync_copy(eidx_hbm.at[row_a], idx_a)
            ga = pltpu.async_copy(g_hbm.at[idx_a.at[0]], rows_a, sem_ga)
            pltpu.sync_copy(eidx_hbm.at[row_a + 1], idx_b)
            gb = pltpu.async_copy(g_hbm.at[idx_b.at[0]], rows_b, sem_gb)
            pltpu.sync_copy(eidx_hbm.at[row_a + 2], idx_c)
            gc = pltpu.async_copy(g_hbm.at[idx_c.at[0]], rows_c, sem_gc)
            ga.wait()
            sa = pltpu.async_copy(rows_a, acc.at[idx_a.at[1]], sem_sa, add=True)
            gb.wait()
            sb = pltpu.async_copy(rows_b, acc.at[idx_b.at[1]], sem_sb, add=True)
            gc.wait()
            sc = pltpu.async_copy(rows_c, acc.at[idx_c.at[1]], sem_sc, add=True)
            sa.wait()
            sb.wait()
            sc.wait()
            return carry

        lax.fori_loop(0, NCH // 3, triple, 0)
        plsc.subcore_barrier()

        bufs = (rows_a, rows_b)
        sems = (sem_ga, sem_gb)
        descs = {}
        chunks = []
        off = 0
        while off < RPT:
            cb = min(CHUNK, RPT - off)
            chunks.append((off, cb))
            off += cb
        for i, (off, cb) in enumerate(chunks):
            if i >= 2:
                descs[i - 2].wait()
            buf = bufs[i % 2]
            pltpu.sync_copy(acc.at[pl.ds(r0 + off, cb)], buf.at[pl.ds(0, cb)])
            descs[i] = pltpu.async_copy(buf.at[pl.ds(0, cb)],
                                        out_hbm.at[c, pl.ds(r0 + off, cb)],
                                        sems[i % 2])
        for i in range(max(0, len(chunks) - 2), len(chunks)):
            descs[i].wait()

    return pl.kernel(
        body,
        out_type=jax.ShapeDtypeStruct((NC, NPAD, d), jnp.float32),
        mesh=_MESH,
        scratch_types=(
            [pltpu.VMEM((2, CHUNK), jnp.int32)] * 3
            + [pltpu.VMEM((CHUNK, d), jnp.float32)] * 3
            + [pltpu.VMEM_SHARED((NPAD, d), jnp.float32)]
            + [pltpu.SemaphoreType.DMA] * 6
        ),
    )


_prop128 = _make_prop(D1)


def _mm1_body(x_ref, w1_ref, h_ref):
    h_ref[...] = jnp.dot(x_ref[...], w1_ref[...],
                         preferred_element_type=jnp.float32)


_mm1 = pl.pallas_call(
    _mm1_body,
    out_shape=jax.ShapeDtypeStruct((NPAD, D1), jnp.float32),
)


def _stage_a_body(h_ref, degp_ref, g_ref, dinv_ref):
    deg0 = jnp.sum(degp_ref[...], axis=1, keepdims=True)
    dinv = lax.rsqrt(jnp.maximum(deg0, 1.0))
    dinv_ref[...] = dinv
    g_ref[...] = h_ref[...] * dinv


_stage_a = pl.pallas_call(
    _stage_a_body,
    out_shape=[
        jax.ShapeDtypeStruct((NPAD, D1), jnp.float32),
        jax.ShapeDtypeStruct((NPAD, 1), jnp.float32),
    ],
)


def _stage_b_body(p_ref, dinv_ref, b1_ref, w2_ref, g2_ref):
    acc = p_ref[0] + p_ref[1]
    dinv = dinv_ref[...]
    h = jnp.maximum(acc * dinv + b1_ref[...], 0.0)
    g2_ref[...] = jnp.dot(h, w2_ref[...],
                          preferred_element_type=jnp.float32) * dinv


_stage_b = pl.pallas_call(
    _stage_b_body,
    out_shape=jax.ShapeDtypeStruct((NPAD, D2), jnp.float32),
)


def _stage_c_body(q_ref, dinv_ref, b2_ref, o_ref):
    acc = q_ref[0] + q_ref[1]
    logits = acc * dinv_ref[...] + b2_ref[...]
    col = lax.broadcasted_iota(jnp.int32, (NPAD, D2), 1)
    valid = col < DC
    logits = jnp.where(valid, logits, -jnp.inf)
    m = jnp.max(logits, axis=1, keepdims=True)
    ex = jnp.where(valid, jnp.exp(logits - m), 0.0)
    lse = jnp.log(jnp.sum(ex, axis=1, keepdims=True))
    out = logits - m - lse
    o_ref[...] = out[:N, :DC]


_stage_c = pl.pallas_call(
    _stage_c_body,
    out_shape=jax.ShapeDtypeStruct((N, DC), jnp.float32),
)


def kernel(inputs, edge_index, W1, b1, W2, b2, epoch):
    ei = edge_index.astype(jnp.int32)
    pad = N + jnp.arange(EPAD - E, dtype=jnp.int32) % (NPAD - N)
    src = jnp.concatenate([ei[0], pad])
    dst = jnp.concatenate([ei[1], pad])
    eidx = jnp.stack([src.reshape(NW * NCH, CHUNK),
                      dst.reshape(NW * NCH, CHUNK)], axis=1)

    x = jnp.concatenate(
        [inputs, jnp.zeros((NPAD - N, D1), jnp.float32)], axis=0)
    w2p = jnp.concatenate(
        [W2, jnp.zeros((D1, D2 - DC), jnp.float32)], axis=1)
    b1r = b1.reshape(1, D1)
    b2r = jnp.concatenate([b2, jnp.zeros((D2 - DC,), jnp.float32)]
                          ).reshape(1, D2)

    z128 = jnp.zeros((NPAD, D1), jnp.float32)

    degp = _deg_kernel(eidx)
    h1 = _mm1(x, W1)
    degt = jnp.transpose(degp[:, 0, :])
    g1, dinv = _stage_a(h1, degt)
    p = _prop128(g1, eidx, z128)
    g2 = _stage_b(p, dinv, b1r, w2p)
    q = _prop128(g2, eidx, z128)
    return _stage_c(q, dinv, b2r)


# speedup vs baseline: 4.2928x; 1.0846x over previous
import jax
import jax.numpy as jnp
from jax import lax
from jax.experimental import pallas as pl
from jax.experimental.pallas import tpu as pltpu
from jax.experimental.pallas import tpu_sc as plsc

N = 10000
E = 320000
D1 = 128
DC = 40
D2 = 64

NC = 2
NS = 16
NW = NC * NS
CHUNK = 128

NPAD = 10112
RPT = NPAD // NS

NCH = 81
EPW = NCH * CHUNK
EPAD = NW * EPW

_MESH = plsc.VectorSubcoreMesh(core_axis_name="c", subcore_axis_name="s")


def _deg_body(eidx_hbm, out_hbm, didx_a, didx_b, deg_v, sem_a, sem_b):
    c = lax.axis_index("c")
    s = lax.axis_index("s")
    wid = s * NC + c
    row0 = wid * NCH

    zero16 = jnp.zeros((16,), jnp.float32)
    ones = jnp.ones((16,), jnp.float32)

    def zb(i, carry):
        deg_v[pl.ds(i * 16, 16)] = zero16
        return carry

    lax.fori_loop(0, NPAD // 16, zb, 0)

    def scat(didx):
        for k in range(CHUNK // 16):
            idx16 = didx[pl.ds(k * 16, 16)]
            plsc.addupdate_scatter(deg_v, [idx16], ones)

    def pair(g, carry):
        j = row0 + 2 * g
        la = pltpu.async_copy(eidx_hbm.at[j, 1], didx_a, sem_a)
        lb = pltpu.async_copy(eidx_hbm.at[j + 1, 1], didx_b, sem_b)
        la.wait()
        scat(didx_a)
        lb.wait()
        scat(didx_b)
        return carry

    lax.fori_loop(0, NCH // 2, pair, 0)
    pltpu.sync_copy(eidx_hbm.at[row0 + NCH - 1, 1], didx_a)
    scat(didx_a)
    pltpu.sync_copy(deg_v, out_hbm.at[wid, 0])


_deg_kernel = pl.kernel(
    _deg_body,
    out_type=jax.ShapeDtypeStruct((NW, 8, NPAD), jnp.float32),
    mesh=_MESH,
    scratch_types=[
        pltpu.VMEM((CHUNK,), jnp.int32),
        pltpu.VMEM((CHUNK,), jnp.int32),
        pltpu.VMEM((NPAD,), jnp.float32),
        pltpu.SemaphoreType.DMA,
        pltpu.SemaphoreType.DMA,
    ],
    compiler_params=pltpu.CompilerParams(needs_layout_passes=False),
)


def _make_prop(d, tc_tiling=True):
    def body(g_hbm, eidx_hbm, z_hbm, out_hbm,
             idx_a, idx_b, idx_c,
             rows_a, rows_b, rows_c, acc,
             sem_ga, sem_gb, sem_gc, sem_sa, sem_sb, sem_sc):
        c = lax.axis_index("c")
        s = lax.axis_index("s")
        wid = s * NC + c
        r0 = s * RPT

        def row_chunks(fn):
            off = 0
            while off < RPT:
                cb = min(CHUNK, RPT - off)
                fn(off, cb)
                off += cb

        def init(off, cb):
            pltpu.sync_copy(z_hbm.at[pl.ds(r0 + off, cb)],
                            rows_a.at[pl.ds(0, cb)])
            pltpu.sync_copy(rows_a.at[pl.ds(0, cb)],
                            acc.at[pl.ds(r0 + off, cb)])

        row_chunks(init)
        plsc.subcore_barrier()

        def triple(g, carry):
            row_a = wid * NCH + 3 * g
            pltpu.sync_copy(eidx_hbm.at[row_a], idx_a)
            ga = pltpu.async_copy(g_hbm.at[idx_a.at[0]], rows_a, sem_ga)
            pltpu.sync_copy(eidx_hbm.at[row_a + 1], idx_b)
            gb = pltpu.async_copy(g_hbm.at[idx_b.at[0]], rows_b, sem_gb)
            pltpu.sync_copy(eidx_hbm.at[row_a + 2], idx_c)
            gc = pltpu.async_copy(g_hbm.at[idx_c.at[0]], rows_c, sem_gc)
            ga.wait()
            sa = pltpu.async_copy(rows_a, acc.at[idx_a.at[1]], sem_sa, add=True)
            gb.wait()
            sb = pltpu.async_copy(rows_b, acc.at[idx_b.at[1]], sem_sb, add=True)
            gc.wait()
            sc = pltpu.async_copy(rows_c, acc.at[idx_c.at[1]], sem_sc, add=True)
            sa.wait()
            sb.wait()
            sc.wait()
            return carry

        lax.fori_loop(0, NCH // 3, triple, 0)
        plsc.subcore_barrier()

        bufs = (rows_a, rows_b)
        sems = (sem_ga, sem_gb)
        descs = {}
        chunks = []
        off = 0
        while off < RPT:
            cb = min(CHUNK, RPT - off)
            chunks.append((off, cb))
            off += cb
        for i, (off, cb) in enumerate(chunks):
            if i >= 2:
                descs[i - 2].wait()
            buf = bufs[i % 2]
            pltpu.sync_copy(acc.at[pl.ds(r0 + off, cb)], buf.at[pl.ds(0, cb)])
            descs[i] = pltpu.async_copy(buf.at[pl.ds(0, cb)],
                                        out_hbm.at[c, pl.ds(r0 + off, cb)],
                                        sems[i % 2])
        for i in range(max(0, len(chunks) - 2), len(chunks)):
            descs[i].wait()

    return pl.kernel(
        body,
        out_type=jax.ShapeDtypeStruct((NC, NPAD, d), jnp.float32),
        mesh=_MESH,
        scratch_types=(
            [pltpu.VMEM((2, CHUNK), jnp.int32)] * 3
            + [pltpu.VMEM((CHUNK, d), jnp.float32)] * 3
            + [pltpu.VMEM_SHARED((NPAD, d), jnp.float32)]
            + [pltpu.SemaphoreType.DMA] * 6
        ),
        compiler_params=(
            None if tc_tiling
            else pltpu.CompilerParams(use_tc_tiling_on_sc=False)),
    )


_prop128 = _make_prop(D1)
_prop64 = _make_prop(D2, tc_tiling=False)


def _mm1_body(x_ref, w1_ref, h_ref):
    h_ref[...] = jnp.dot(x_ref[...], w1_ref[...],
                         preferred_element_type=jnp.float32)


_mm1 = pl.pallas_call(
    _mm1_body,
    out_shape=jax.ShapeDtypeStruct((NPAD, D1), jnp.float32),
)


def _stage_a_body(h_ref, degp_ref, g_ref, dinv_ref):
    deg0 = jnp.sum(degp_ref[...], axis=1, keepdims=True)
    dinv = lax.rsqrt(jnp.maximum(deg0, 1.0))
    dinv_ref[...] = dinv
    g_ref[...] = h_ref[...] * dinv


_stage_a = pl.pallas_call(
    _stage_a_body,
    out_shape=[
        jax.ShapeDtypeStruct((NPAD, D1), jnp.float32),
        jax.ShapeDtypeStruct((NPAD, 1), jnp.float32),
    ],
)


def _stage_b_body(p_ref, dinv_ref, b1_ref, w2_ref, g2_ref):
    acc = p_ref[0] + p_ref[1]
    dinv = dinv_ref[...]
    h = jnp.maximum(acc * dinv + b1_ref[...], 0.0)
    g2_ref[...] = jnp.dot(h, w2_ref[...],
                          preferred_element_type=jnp.float32) * dinv


_stage_b = pl.pallas_call(
    _stage_b_body,
    out_shape=jax.ShapeDtypeStruct((NPAD, D2), jnp.float32),
)


def _stage_c_body(q_ref, dinv_ref, b2_ref, o_ref):
    acc = q_ref[0] + q_ref[1]
    logits = acc * dinv_ref[...] + b2_ref[...]
    col = lax.broadcasted_iota(jnp.int32, (NPAD, D2), 1)
    valid = col < DC
    logits = jnp.where(valid, logits, -jnp.inf)
    m = jnp.max(logits, axis=1, keepdims=True)
    ex = jnp.where(valid, jnp.exp(logits - m), 0.0)
    lse = jnp.log(jnp.sum(ex, axis=1, keepdims=True))
    out = logits - m - lse
    o_ref[...] = out[:N, :DC]


_stage_c = pl.pallas_call(
    _stage_c_body,
    out_shape=jax.ShapeDtypeStruct((N, DC), jnp.float32),
)


def kernel(inputs, edge_index, W1, b1, W2, b2, epoch):
    ei = edge_index.astype(jnp.int32)
    pad = N + jnp.arange(EPAD - E, dtype=jnp.int32) % (NPAD - N)
    src = jnp.concatenate([ei[0], pad])
    dst = jnp.concatenate([ei[1], pad])
    eidx = jnp.stack([src.reshape(NW * NCH, CHUNK),
                      dst.reshape(NW * NCH, CHUNK)], axis=1)

    x = jnp.concatenate(
        [inputs, jnp.zeros((NPAD - N, D1), jnp.float32)], axis=0)
    w2p = jnp.concatenate(
        [W2, jnp.zeros((D1, D2 - DC), jnp.float32)], axis=1)
    b1r = b1.reshape(1, D1)
    b2r = jnp.concatenate([b2, jnp.zeros((D2 - DC,), jnp.float32)]
                          ).reshape(1, D2)

    z128 = jnp.zeros((NPAD, D1), jnp.float32)
    z64 = jnp.zeros((NPAD, D2), jnp.float32)

    degp = _deg_kernel(eidx)
    h1 = _mm1(x, W1)
    degt = jnp.transpose(degp[:, 0, :])
    g1, dinv = _stage_a(h1, degt)
    p = _prop128(g1, eidx, z128)
    g2 = _stage_b(p, dinv, b1r, w2p)
    q = _prop64(g2, eidx, z64)
    return _stage_c(q, dinv, b2r)
